# Initial kernel scaffold; baseline (speedup 1.0000x reference)
import jax
import jax.numpy as jnp
from jax.experimental import pallas as pl


def kernel(x, edge, w1, b1, w2, b2):
    raise NotImplementedError("write your pallas kernel here")


# speedup vs baseline: 6.2046x; 6.2046x over previous
import functools

import jax
import jax.numpy as jnp
from jax import lax
from jax.experimental import pallas as pl
from jax.experimental.pallas import tpu as pltpu
from jax.experimental.pallas import tpu_sc as plsc

_NC = 2
_NS = 16
_C = 128
_G = 4

_N = 10000
_D = 128
_DH = _D // _NC
_NACC = 10240
_ZROWS = _NACC // _NS

_BLK = 2000


def _mesh():
    return plsc.VectorSubcoreMesh(core_axis_name="c", subcore_axis_name="s")


def _make_sc_prop(kp):
    @functools.partial(
        pl.kernel,
        out_type=jax.ShapeDtypeStruct((_NC * _NACC, _DH), jnp.float32),
        mesh=_mesh(),
        scratch_types=[
            pltpu.VMEM((kp, _C), jnp.int32),
            pltpu.VMEM((kp, _C), jnp.int32),
            pltpu.VMEM((_G * _C, _DH), jnp.float32),
            pltpu.VMEM_SHARED((_NACC, _DH), jnp.float32),
            pltpu.SemaphoreType.DMA,
        ],
        compiler_params=pltpu.CompilerParams(use_tc_tiling_on_sc=False),
    )
    def sc_prop(tab_hbm, srcs_hbm, dsts_hbm, zeros_hbm, out_hbm,
                src_v, dst_v, rows_v, acc_sh, sem):
        cid = lax.axis_index("c")
        sid = lax.axis_index("s")

        pltpu.sync_copy(zeros_hbm.at[pl.ds(sid * _ZROWS, _ZROWS)],
                        acc_sh.at[pl.ds(sid * _ZROWS, _ZROWS)])
        pltpu.sync_copy(srcs_hbm.at[pl.ds((cid * _NS + sid) * kp, kp)], src_v)
        pltpu.sync_copy(dsts_hbm.at[pl.ds(sid * kp, kp)], dst_v)
        plsc.subcore_barrier()

        def group(gi, carry):
            base = gi * _G
            waits = []
            for q in range(_G):
                waits.append(pltpu.async_copy(
                    tab_hbm.at[src_v.at[base + q]],
                    rows_v.at[pl.ds(q * _C, _C)], sem))
            for w in waits:
                w.wait()
            for q in range(_G):
                pltpu.sync_copy(rows_v.at[pl.ds(q * _C, _C)],
                                acc_sh.at[dst_v.at[base + q]], add=True)
            return carry

        lax.fori_loop(0, kp // _G, group, 0)
        plsc.subcore_barrier()

        pltpu.sync_copy(
            acc_sh.at[pl.ds(sid * _ZROWS, _ZROWS)],
            out_hbm.at[pl.ds(cid * _NACC + sid * _ZROWS, _ZROWS)])

    return sc_prop


def _make_sc_deg(kd):
    @functools.partial(
        pl.kernel,
        out_type=jax.ShapeDtypeStruct((_NC * _NACC, 16), jnp.float32),
        mesh=_mesh(),
        scratch_types=[
            pltpu.VMEM((kd, _C), jnp.int32),
            pltpu.VMEM((_C, 16), jnp.float32),
            pltpu.VMEM_SHARED((_NACC, 16), jnp.float32),
        ],
        compiler_params=pltpu.CompilerParams(use_tc_tiling_on_sc=False),
    )
    def sc_deg(srcs_hbm, zeros_hbm, ones_hbm, out_hbm,
               src_v, ones_v, acc_sh):
        cid = lax.axis_index("c")
        sid = lax.axis_index("s")
        wid = sid * _NC + cid

        pltpu.sync_copy(zeros_hbm.at[pl.ds(sid * _ZROWS, _ZROWS)],
                        acc_sh.at[pl.ds(sid * _ZROWS, _ZROWS)])
        pltpu.sync_copy(ones_hbm, ones_v)
        pltpu.sync_copy(srcs_hbm.at[pl.ds(wid * kd, kd)], src_v)
        plsc.subcore_barrier()

        def chunk(j, carry):
            pltpu.sync_copy(ones_v, acc_sh.at[src_v.at[j]], add=True)
            return carry

        lax.fori_loop(0, kd, chunk, 0)
        plsc.subcore_barrier()

        pltpu.sync_copy(
            acc_sh.at[pl.ds(sid * _ZROWS, _ZROWS)],
            out_hbm.at[pl.ds(cid * _NACC + sid * _ZROWS, _ZROWS)])

    return sc_deg


def _prep_body(dp_ref, x_ref, dis_ref, g0_ref):
    dp = dp_ref[...]
    deg = dp[0] + dp[1]
    dis = jnp.where(deg > 0, lax.rsqrt(jnp.where(deg > 0, deg, 1.0)), 0.0)
    dis_ref[...] = dis
    x = x_ref[...]
    dis_c = dis[:, 0:1]
    g0_ref[...] = jnp.stack([dis_c * x[:, :_DH], dis_c * x[:, _DH:]])


def _tc_prep(deg_parts, x):
    grid = _N // _BLK
    return pl.pallas_call(
        _prep_body,
        grid=(grid,),
        in_specs=[
            pl.BlockSpec((_NC, _BLK, 16), lambda i: (0, i, 0)),
            pl.BlockSpec((_BLK, _D), lambda i: (i, 0)),
        ],
        out_specs=[
            pl.BlockSpec((_BLK, 16), lambda i: (i, 0)),
            pl.BlockSpec((_NC, _BLK, _DH), lambda i: (0, i, 0)),
        ],
        out_shape=[
            jax.ShapeDtypeStruct((_N, 16), jnp.float32),
            jax.ShapeDtypeStruct((_NC, _N, _DH), jnp.float32),
        ],
    )(deg_parts, x)


def _combine_body(s0_ref, s1_ref, dis_ref, p_ref, g_ref):
    dis_c = dis_ref[...][:, 0:1]
    p0 = (-dis_c) * s0_ref[...]
    p1 = (-dis_c) * s1_ref[...]
    p_ref[...] = jnp.concatenate([p0, p1], axis=1)
    g_ref[...] = jnp.stack([dis_c * p0, dis_c * p1])


def _tc_combine(s0, s1, dis):
    grid = _N // _BLK
    return pl.pallas_call(
        _combine_body,
        grid=(grid,),
        in_specs=[
            pl.BlockSpec((_BLK, _DH), lambda i: (i, 0)),
            pl.BlockSpec((_BLK, _DH), lambda i: (i, 0)),
            pl.BlockSpec((_BLK, 16), lambda i: (i, 0)),
        ],
        out_specs=[
            pl.BlockSpec((_BLK, _D), lambda i: (i, 0)),
            pl.BlockSpec((_NC, _BLK, _DH), lambda i: (0, i, 0)),
        ],
        out_shape=[
            jax.ShapeDtypeStruct((_N, _D), jnp.float32),
            jax.ShapeDtypeStruct((_NC, _N, _DH), jnp.float32),
        ],
    )(s0, s1, dis)


def _dense_body(relu, h_ref, p1_ref, p2_ref, w_ref, b_ref, dis_ref,
                out_ref, g_ref):
    h = h_ref[...]
    acc = jnp.dot(h, w_ref[0], preferred_element_type=jnp.float32)
    acc += jnp.dot(p1_ref[...], w_ref[1], preferred_element_type=jnp.float32)
    acc += jnp.dot(2.0 * p2_ref[...] - h, w_ref[2],
                   preferred_element_type=jnp.float32)
    acc += b_ref[...]
    if relu:
        acc = jnp.maximum(acc, 0.0)
    out_ref[...] = acc
    dis_c = dis_ref[...][:, 0:1]
    g_ref[...] = jnp.stack([dis_c * acc[:, :_DH], dis_c * acc[:, _DH:]])


def _tc_dense(h, p1, p2, w, b, dis, relu):
    grid = _N // _BLK
    return pl.pallas_call(
        functools.partial(_dense_body, relu),
        grid=(grid,),
        in_specs=[
            pl.BlockSpec((_BLK, _D), lambda i: (i, 0)),
            pl.BlockSpec((_BLK, _D), lambda i: (i, 0)),
            pl.BlockSpec((_BLK, _D), lambda i: (i, 0)),
            pl.BlockSpec((3, _D, _D), lambda i: (0, 0, 0)),
            pl.BlockSpec((1, _D), lambda i: (0, 0)),
            pl.BlockSpec((_BLK, 16), lambda i: (i, 0)),
        ],
        out_specs=[
            pl.BlockSpec((_BLK, _D), lambda i: (i, 0)),
            pl.BlockSpec((_NC, _BLK, _DH), lambda i: (0, i, 0)),
        ],
        out_shape=[
            jax.ShapeDtypeStruct((_N, _D), jnp.float32),
            jax.ShapeDtypeStruct((_NC, _N, _DH), jnp.float32),
        ],
    )(h, p1, p2, w, b.reshape(1, _D), dis)


def kernel(x, edge, w1, b1, w2, b2):
    n, d = x.shape
    e = edge.shape[1]
    src = edge[0].astype(jnp.int32)
    dst = edge[1].astype(jnp.int32)

    kd = (-(-e // (_NC * _NS * _C)) + 7) // 8 * 8
    pad_d = _NC * _NS * kd * _C - e
    src_deg = jnp.concatenate(
        [src, jnp.full((pad_d,), n, jnp.int32)]).reshape(_NC * _NS * kd, _C)

    kp = (-(-e // (_NS * _C)) + 7) // 8 * 8
    pad_p = _NS * kp * _C - e
    src_p = jnp.concatenate([src, jnp.zeros((pad_p,), jnp.int32)])
    src_fs = jnp.concatenate(
        [src_p, src_p + jnp.int32(n)]).reshape(_NC * _NS * kp, _C)
    dst_fs = jnp.concatenate(
        [dst, jnp.full((pad_p,), n, jnp.int32)]).reshape(_NS * kp, _C)

    zeros_h = jnp.zeros((_NACC, _DH), jnp.float32)
    zeros16 = jnp.zeros((_NACC, 16), jnp.float32)
    ones16 = jnp.ones((_C, 16), jnp.float32)

    sc_deg = _make_sc_deg(kd)
    sc_prop = _make_sc_prop(kp)

    deg_parts = sc_deg(src_deg, zeros16, ones16)
    deg_parts = deg_parts.reshape(_NC, _NACC, 16)[:, :n, :]
    dis, g0 = _tc_prep(deg_parts, x)

    def prop_halves(g):
        s = sc_prop(g.reshape(_NC * n, _DH), src_fs, dst_fs, zeros_h)
        s = s.reshape(_NC, _NACC, _DH)
        return s[0, :n, :], s[1, :n, :]

    p1, g1 = _tc_combine(*prop_halves(g0), dis)
    p2, _ = _tc_combine(*prop_halves(g1), dis)
    out1, g2 = _tc_dense(x, p1, p2, w1, b1, dis, relu=True)
    p3, g3 = _tc_combine(*prop_halves(g2), dis)
    p4, _ = _tc_combine(*prop_halves(g3), dis)
    out, _ = _tc_dense(out1, p3, p4, w2, b2, dis, relu=False)
    return out


# speedup vs baseline: 6.8665x; 1.1067x over previous
import functools

import jax
import jax.numpy as jnp
from jax import lax
from jax.experimental import pallas as pl
from jax.experimental.pallas import tpu as pltpu
from jax.experimental.pallas import tpu_sc as plsc

_NC = 2
_NS = 16
_C = 128
_G = 2

_N = 10000
_D = 128
_DH = _D // _NC
_NACC = 10240
_ZROWS = _NACC // _NS

_BLK = 2000


def _mesh():
    return plsc.VectorSubcoreMesh(core_axis_name="c", subcore_axis_name="s")


def _make_sc_prop(kp):
    ngroups = kp // _G

    @functools.partial(
        pl.kernel,
        out_type=jax.ShapeDtypeStruct((_NC * _NACC, _DH), jnp.float32),
        mesh=_mesh(),
        scratch_types=[
            pltpu.VMEM((kp, _C), jnp.int32),
            pltpu.VMEM((kp, _C), jnp.int32),
            pltpu.VMEM((_G * _C, _DH), jnp.float32),
            pltpu.VMEM((_G * _C, _DH), jnp.float32),
            pltpu.VMEM_SHARED((_NACC, _DH), jnp.float32),
            pltpu.SemaphoreType.DMA,
            pltpu.SemaphoreType.DMA,
            pltpu.SemaphoreType.DMA,
            pltpu.SemaphoreType.DMA,
        ],
        compiler_params=pltpu.CompilerParams(use_tc_tiling_on_sc=False),
    )
    def sc_prop(tab_hbm, srcs_hbm, dsts_hbm, zeros_hbm, out_hbm,
                src_v, dst_v, rows_a, rows_b, acc_sh,
                sem_ga, sem_gb, sem_sa, sem_sb):
        cid = lax.axis_index("c")
        sid = lax.axis_index("s")

        pltpu.sync_copy(zeros_hbm.at[pl.ds(sid * _ZROWS, _ZROWS)],
                        acc_sh.at[pl.ds(sid * _ZROWS, _ZROWS)])
        pltpu.sync_copy(srcs_hbm.at[pl.ds((cid * _NS + sid) * kp, kp)], src_v)
        pltpu.sync_copy(dsts_hbm.at[pl.ds(sid * kp, kp)], dst_v)
        plsc.subcore_barrier()

        def fire_gather(grp, buf, sem):
            base = grp * _G
            for q in range(_G):
                pltpu.async_copy(tab_hbm.at[src_v.at[base + q]],
                                 buf.at[pl.ds(q * _C, _C)], sem)

        def drain(buf, sem):
            for q in range(_G):
                pltpu.make_async_copy(tab_hbm.at[src_v.at[q]],
                                      buf.at[pl.ds(q * _C, _C)], sem).wait()

        def fire_scatter(grp, buf, sem):
            base = grp * _G
            for q in range(_G):
                pltpu.async_copy(buf.at[pl.ds(q * _C, _C)],
                                 acc_sh.at[dst_v.at[base + q]], sem, add=True)

        def drain_scatter(buf, sem):
            for q in range(_G):
                pltpu.make_async_copy(buf.at[pl.ds(q * _C, _C)],
                                      acc_sh.at[dst_v.at[q]], sem).wait()

        fire_gather(0, rows_a, sem_ga)
        fire_gather(1, rows_b, sem_gb)

        def pipe(i, carry):
            t = 2 * i
            drain(rows_a, sem_ga)
            fire_scatter(t, rows_a, sem_sa)
            drain(rows_b, sem_gb)
            fire_scatter(t + 1, rows_b, sem_sb)
            drain_scatter(rows_a, sem_sa)
            fire_gather(lax.rem(t + 2, ngroups), rows_a, sem_ga)
            drain_scatter(rows_b, sem_sb)
            fire_gather(lax.rem(t + 3, ngroups), rows_b, sem_gb)
            return carry

        lax.fori_loop(0, ngroups // 2, pipe, 0)
        drain(rows_a, sem_ga)
        drain(rows_b, sem_gb)
        plsc.subcore_barrier()

        pltpu.sync_copy(
            acc_sh.at[pl.ds(sid * _ZROWS, _ZROWS)],
            out_hbm.at[pl.ds(cid * _NACC + sid * _ZROWS, _ZROWS)])

    return sc_prop


def _make_sc_deg(kd):
    @functools.partial(
        pl.kernel,
        out_type=jax.ShapeDtypeStruct((_NC * _NACC, 16), jnp.float32),
        mesh=_mesh(),
        scratch_types=[
            pltpu.VMEM((kd, _C), jnp.int32),
            pltpu.VMEM((_C, 16), jnp.float32),
            pltpu.VMEM_SHARED((_NACC, 16), jnp.float32),
        ],
        compiler_params=pltpu.CompilerParams(use_tc_tiling_on_sc=False),
    )
    def sc_deg(srcs_hbm, zeros_hbm, ones_hbm, out_hbm,
               src_v, ones_v, acc_sh):
        cid = lax.axis_index("c")
        sid = lax.axis_index("s")
        wid = sid * _NC + cid

        pltpu.sync_copy(zeros_hbm.at[pl.ds(sid * _ZROWS, _ZROWS)],
                        acc_sh.at[pl.ds(sid * _ZROWS, _ZROWS)])
        pltpu.sync_copy(ones_hbm, ones_v)
        pltpu.sync_copy(srcs_hbm.at[pl.ds(wid * kd, kd)], src_v)
        plsc.subcore_barrier()

        def chunk(j, carry):
            pltpu.sync_copy(ones_v, acc_sh.at[src_v.at[j]], add=True)
            return carry

        lax.fori_loop(0, kd, chunk, 0)
        plsc.subcore_barrier()

        pltpu.sync_copy(
            acc_sh.at[pl.ds(sid * _ZROWS, _ZROWS)],
            out_hbm.at[pl.ds(cid * _NACC + sid * _ZROWS, _ZROWS)])

    return sc_deg


def _prep_body(dp_ref, x_ref, dis_ref, g0_ref):
    dp = dp_ref[...]
    deg = dp[0] + dp[1]
    dis = jnp.where(deg > 0, lax.rsqrt(jnp.where(deg > 0, deg, 1.0)), 0.0)
    dis_ref[...] = dis
    x = x_ref[...]
    dis_c = dis[:, 0:1]
    g0_ref[...] = jnp.stack([dis_c * x[:, :_DH], dis_c * x[:, _DH:]])


def _tc_prep(deg_parts, x):
    grid = _N // _BLK
    return pl.pallas_call(
        _prep_body,
        grid=(grid,),
        in_specs=[
            pl.BlockSpec((_NC, _BLK, 16), lambda i: (0, i, 0)),
            pl.BlockSpec((_BLK, _D), lambda i: (i, 0)),
        ],
        out_specs=[
            pl.BlockSpec((_BLK, 16), lambda i: (i, 0)),
            pl.BlockSpec((_NC, _BLK, _DH), lambda i: (0, i, 0)),
        ],
        out_shape=[
            jax.ShapeDtypeStruct((_N, 16), jnp.float32),
            jax.ShapeDtypeStruct((_NC, _N, _DH), jnp.float32),
        ],
    )(deg_parts, x)


def _combine_body(s0_ref, s1_ref, dis_ref, p_ref, g_ref):
    dis_c = dis_ref[...][:, 0:1]
    p0 = (-dis_c) * s0_ref[...]
    p1 = (-dis_c) * s1_ref[...]
    p_ref[...] = jnp.concatenate([p0, p1], axis=1)
    g_ref[...] = jnp.stack([dis_c * p0, dis_c * p1])


def _tc_combine(s0, s1, dis):
    grid = _N // _BLK
    return pl.pallas_call(
        _combine_body,
        grid=(grid,),
        in_specs=[
            pl.BlockSpec((_BLK, _DH), lambda i: (i, 0)),
            pl.BlockSpec((_BLK, _DH), lambda i: (i, 0)),
            pl.BlockSpec((_BLK, 16), lambda i: (i, 0)),
        ],
        out_specs=[
            pl.BlockSpec((_BLK, _D), lambda i: (i, 0)),
            pl.BlockSpec((_NC, _BLK, _DH), lambda i: (0, i, 0)),
        ],
        out_shape=[
            jax.ShapeDtypeStruct((_N, _D), jnp.float32),
            jax.ShapeDtypeStruct((_NC, _N, _DH), jnp.float32),
        ],
    )(s0, s1, dis)


def _dense_body(relu, h_ref, p1_ref, s0_ref, s1_ref, w_ref, b_ref, dis_ref,
                out_ref, g_ref):
    h = h_ref[...]
    nd = -dis_ref[...][:, 0:1]
    p2 = jnp.concatenate([nd * s0_ref[...], nd * s1_ref[...]], axis=1)
    acc = jnp.dot(h, w_ref[0], preferred_element_type=jnp.float32)
    acc += jnp.dot(p1_ref[...], w_ref[1], preferred_element_type=jnp.float32)
    acc += jnp.dot(2.0 * p2 - h, w_ref[2],
                   preferred_element_type=jnp.float32)
    acc += b_ref[...]
    if relu:
        acc = jnp.maximum(acc, 0.0)
    out_ref[...] = acc
    dis_c = dis_ref[...][:, 0:1]
    g_ref[...] = jnp.stack([dis_c * acc[:, :_DH], dis_c * acc[:, _DH:]])


def _tc_dense(h, p1, s0, s1, w, b, dis, relu):
    grid = _N // _BLK
    return pl.pallas_call(
        functools.partial(_dense_body, relu),
        grid=(grid,),
        in_specs=[
            pl.BlockSpec((_BLK, _D), lambda i: (i, 0)),
            pl.BlockSpec((_BLK, _D), lambda i: (i, 0)),
            pl.BlockSpec((_BLK, _DH), lambda i: (i, 0)),
            pl.BlockSpec((_BLK, _DH), lambda i: (i, 0)),
            pl.BlockSpec((3, _D, _D), lambda i: (0, 0, 0)),
            pl.BlockSpec((1, _D), lambda i: (0, 0)),
            pl.BlockSpec((_BLK, 16), lambda i: (i, 0)),
        ],
        out_specs=[
            pl.BlockSpec((_BLK, _D), lambda i: (i, 0)),
            pl.BlockSpec((_NC, _BLK, _DH), lambda i: (0, i, 0)),
        ],
        out_shape=[
            jax.ShapeDtypeStruct((_N, _D), jnp.float32),
            jax.ShapeDtypeStruct((_NC, _N, _DH), jnp.float32),
        ],
    )(h, p1, s0, s1, w, b.reshape(1, _D), dis)


def kernel(x, edge, w1, b1, w2, b2):
    n, d = x.shape
    e = edge.shape[1]
    src = edge[0].astype(jnp.int32)
    dst = edge[1].astype(jnp.int32)

    kd = (-(-e // (_NC * _NS * _C)) + 7) // 8 * 8
    pad_d = _NC * _NS * kd * _C - e
    src_deg = jnp.concatenate(
        [src, jnp.full((pad_d,), n, jnp.int32)]).reshape(_NC * _NS * kd, _C)

    kp = (-(-e // (_NS * _C)) + 7) // 8 * 8
    pad_p = _NS * kp * _C - e
    src_p = jnp.concatenate([src, jnp.zeros((pad_p,), jnp.int32)])
    src_fs = jnp.concatenate(
        [src_p, src_p + jnp.int32(n)]).reshape(_NC * _NS * kp, _C)
    dst_fs = jnp.concatenate(
        [dst, jnp.full((pad_p,), n, jnp.int32)]).reshape(_NS * kp, _C)

    zeros_h = jnp.zeros((_NACC, _DH), jnp.float32)
    zeros16 = jnp.zeros((_NACC, 16), jnp.float32)
    ones16 = jnp.ones((_C, 16), jnp.float32)

    sc_deg = _make_sc_deg(kd)
    sc_prop = _make_sc_prop(kp)

    deg_parts = sc_deg(src_deg, zeros16, ones16)
    deg_parts = deg_parts.reshape(_NC, _NACC, 16)[:, :n, :]
    dis, g0 = _tc_prep(deg_parts, x)

    def prop_halves(g):
        s = sc_prop(g.reshape(_NC * n, _DH), src_fs, dst_fs, zeros_h)
        s = s.reshape(_NC, _NACC, _DH)
        return s[0, :n, :], s[1, :n, :]

    p1, g1 = _tc_combine(*prop_halves(g0), dis)
    s2_0, s2_1 = prop_halves(g1)
    out1, g2 = _tc_dense(x, p1, s2_0, s2_1, w1, b1, dis, relu=True)
    p3, g3 = _tc_combine(*prop_halves(g2), dis)
    s4_0, s4_1 = prop_halves(g3)
    out, _ = _tc_dense(out1, p3, s4_0, s4_1, w2, b2, dis, relu=False)
    return out
